# Initial kernel scaffold; baseline (speedup 1.0000x reference)
import jax
import jax.numpy as jnp
from jax.experimental import pallas as pl


def kernel(x, adj, W, b):
    raise NotImplementedError("write your pallas kernel here")


# speedup vs baseline: 1.4172x; 1.4172x over previous
import functools

import jax
import jax.numpy as jnp
from jax import lax
from jax.experimental import pallas as pl
from jax.experimental.pallas import tpu as pltpu
from jax.experimental.pallas import tpu_sc as plsc

N = 10000
K = 32
D = 128
D_OUT = 128

NC = 2
NS = 16
NW = NC * NS
N_PAD = 10240
NODES_PW = N_PAD // NW
CHUNK = 4
ROWS_PC = CHUNK * K
NCH = NODES_PW // CHUNK
NV = D // 16


def _sc_body(x_hbm, adj_hbm, out_hbm, idx_v, rows0, rows1, acc_v, sem0, sem1):
    cc = lax.axis_index("c")
    ss = lax.axis_index("s")
    wid = ss * NC + cc
    pltpu.sync_copy(adj_hbm.at[pl.ds(wid * NCH, NCH)], idx_v)
    pltpu.async_copy(x_hbm.at[idx_v.at[0]], rows0, sem0)
    pltpu.async_copy(x_hbm.at[idx_v.at[1]], rows1, sem1)

    def reduce_chunk(buf, ch):
        for nloc in range(CHUNK):
            base = nloc * K
            init = tuple(buf[base, pl.ds(d * 16, 16)] for d in range(NV))

            def kbody(kk, accs):
                return tuple(
                    accs[d] + buf[base + kk, pl.ds(d * 16, 16)]
                    for d in range(NV)
                )

            accs = lax.fori_loop(1, K, kbody, init)
            for d in range(NV):
                acc_v[nloc, pl.ds(d * 16, 16)] = accs[d] * (1.0 / K)
        pltpu.sync_copy(
            acc_v, out_hbm.at[pl.ds(wid * NODES_PW + ch * CHUNK, CHUNK)]
        )

    def outer(g, carry):
        for b, (buf, sem) in enumerate(((rows0, sem0), (rows1, sem1))):
            ch = g * 2 + b
            pltpu.make_async_copy(x_hbm.at[idx_v.at[ch]], buf, sem).wait()
            reduce_chunk(buf, ch)

            @pl.when(ch + 2 < NCH)
            def _():
                pltpu.async_copy(x_hbm.at[idx_v.at[ch + 2]], buf, sem)

        return carry

    lax.fori_loop(0, NCH // 2, outer, 0)


def _sc_gather_mean(x, adj_rows):
    mesh = plsc.VectorSubcoreMesh(core_axis_name="c", subcore_axis_name="s")
    f = functools.partial(
        pl.kernel,
        mesh=mesh,
        out_type=jax.ShapeDtypeStruct((N_PAD, D), jnp.float32),
        scratch_types=[
            pltpu.VMEM((NCH, ROWS_PC), jnp.int32),
            pltpu.VMEM((ROWS_PC, D), jnp.float32),
            pltpu.VMEM((ROWS_PC, D), jnp.float32),
            pltpu.VMEM((CHUNK, D), jnp.float32),
            pltpu.SemaphoreType.DMA,
            pltpu.SemaphoreType.DMA,
        ],
    )(_sc_body)
    return f(x, adj_rows)


BM = 1000


def _linear_body(x_ref, agg_ref, w_ref, b_ref, o_ref):
    wt = w_ref[0:D, :]
    wb = w_ref[D : 2 * D, :]
    o_ref[...] = (
        jnp.dot(x_ref[...], wt, preferred_element_type=jnp.float32)
        + jnp.dot(agg_ref[...], wb, preferred_element_type=jnp.float32)
        + b_ref[...]
    )


def _tc_linear(x, agg, W, b):
    return pl.pallas_call(
        _linear_body,
        grid=(N // BM,),
        in_specs=[
            pl.BlockSpec((BM, D), lambda i: (i, 0)),
            pl.BlockSpec((BM, D), lambda i: (i, 0)),
            pl.BlockSpec((2 * D, D_OUT), lambda i: (0, 0)),
            pl.BlockSpec((1, D_OUT), lambda i: (0, 0)),
        ],
        out_specs=pl.BlockSpec((BM, D_OUT), lambda i: (i, 0)),
        out_shape=jax.ShapeDtypeStruct((N, D_OUT), jnp.float32),
    )(x, agg, W, b.reshape(1, D_OUT))


def kernel(x, adj, W, b):
    adj_rows = jnp.pad(adj, ((0, N_PAD - N), (0, 0))).reshape(
        N_PAD // CHUNK, ROWS_PC
    )
    agg = _sc_gather_mean(x, adj_rows)[:N]
    return _tc_linear(x, agg, W, b)
